# baseline (speedup 1.0000x reference)
import jax
import jax.numpy as jnp
from jax import lax
from jax.experimental import pallas as pl
from jax.experimental.pallas import tpu as pltpu
from jax.experimental.pallas import tpu_sc as plsc

BATCH = 16384
D = 32
NC = 2
NS = 16
NW = NC * NS
BPW = BATCH // NW
L = 16
G = BPW // L


CHUNK = 128
NCHUNK = BPW // CHUNK
PACK = 4


def _mf_body(user_hbm, item_hbm, mu_hbm, ub_hbm, ib_hbm, ufac_hbm, ifac_hbm,
             out_hbm, uidx, iidx, umac, imac, uf, itf, ub, ib, outv, muv,
             sem0, sem1):
    wid = lax.axis_index("s") * NC + lax.axis_index("c")
    base_row = wid * NCHUNK
    sems = (sem0, sem1)

    pltpu.sync_copy(user_hbm.at[pl.ds(base_row, NCHUNK)], uidx)
    pltpu.sync_copy(item_hbm.at[pl.ds(base_row, NCHUNK)], iidx)
    pltpu.sync_copy(mu_hbm, muv)

    for j in range(NCHUNK):
        for g in range(CHUNK // L):
            s = pl.ds(g * L, L)
            umac[j, s] = jnp.bitwise_and(uidx[j, s], (1 << USHIFT) - 1)
            imac[j, s] = jnp.bitwise_and(iidx[j, s], (1 << ISHIFT) - 1)

    def start(j):
        slot = j % 2
        cu = pltpu.async_copy(ufac_hbm.at[umac.at[j]],
                              uf.at[pl.ds(slot * CHUNK, CHUNK)], sems[slot])
        ci = pltpu.async_copy(ifac_hbm.at[imac.at[j]],
                              itf.at[pl.ds(slot * CHUNK, CHUNK)], sems[slot])
        cb = pltpu.async_copy(ub_hbm.at[uidx.at[j]],
                              ub.at[pl.ds(j * CHUNK, CHUNK)], sems[slot])
        db = pltpu.async_copy(ib_hbm.at[iidx.at[j]],
                              ib.at[pl.ds(j * CHUNK, CHUNK)], sems[slot])
        return (cu, ci, cb, db)

    mu_v = muv[...]
    inflight = start(0)
    for j in range(NCHUNK):
        for c in inflight:
            c.wait()
        if j + 1 < NCHUNK:
            nxt = start(j + 1)
        slot = j % 2

        def gbody(g, carry):
            rv = slot * CHUNK + g * L + lax.iota(jnp.int32, L)
            s = pl.ds(g * L, L)
            cu = jnp.left_shift(jnp.right_shift(uidx[j, s], USHIFT), 5)
            ci = jnp.left_shift(jnp.right_shift(iidx[j, s], ISHIFT), 5)
            ju = jnp.bitwise_and(cu, 127)
            ji = jnp.bitwise_and(ci, 127)
            shu = jnp.left_shift(jnp.right_shift(cu, 7), 4)
            shi = jnp.left_shift(jnp.right_shift(ci, 7), 4)
            o = pl.ds(j * CHUNK + g * L, L)
            acc = ub[o] + ib[o] + mu_v
            for f in range(D):
                wu = plsc.load_gather(uf, [rv, ju + f])
                wi = plsc.load_gather(itf, [rv, ji + f])
                vu = plsc.bitcast(
                    jnp.left_shift(jnp.right_shift(wu, shu), 16), jnp.float32)
                vi = plsc.bitcast(
                    jnp.left_shift(jnp.right_shift(wi, shi), 16), jnp.float32)
                acc = acc + vu * vi
            outv[o] = acc
            return carry

        lax.fori_loop(0, CHUNK // L, gbody, 0)
        if j + 1 < NCHUNK:
            inflight = nxt

    pltpu.sync_copy(outv, out_hbm.at[pl.ds(wid * BPW, BPW)])


@jax.jit
def _mf(user2, item2, mu, ub_t, ib_t, ufac4, ifac4):
    mesh = plsc.VectorSubcoreMesh(core_axis_name="c", subcore_axis_name="s")
    f = pl.kernel(
        _mf_body,
        out_type=jax.ShapeDtypeStruct((BATCH,), jnp.float32),
        mesh=mesh,
        compiler_params=pltpu.CompilerParams(
            needs_layout_passes=False, use_tc_tiling_on_sc=False),
        scratch_types=[
            pltpu.VMEM((NCHUNK, CHUNK), jnp.int32),
            pltpu.VMEM((NCHUNK, CHUNK), jnp.int32),
            pltpu.VMEM((NCHUNK, CHUNK), jnp.int32),
            pltpu.VMEM((NCHUNK, CHUNK), jnp.int32),
            pltpu.VMEM((2 * CHUNK, 4 * D), jnp.int32),
            pltpu.VMEM((2 * CHUNK, 4 * D), jnp.int32),
            pltpu.VMEM((BPW,), jnp.float32),
            pltpu.VMEM((BPW,), jnp.float32),
            pltpu.VMEM((BPW,), jnp.float32),
            pltpu.VMEM((L,), jnp.float32),
            pltpu.SemaphoreType.DMA,
            pltpu.SemaphoreType.DMA,
        ],
    )
    return f(user2, item2, mu, ub_t, ib_t, ufac4, ifac4)


BLK_W = 16384
NBAND = 8
USHIFT = 17
ISHIFT = 14


def _retile_body(*refs):
    srcs, dst_ref = refs[:-1], refs[-1]
    x = jnp.concatenate([s[...] for s in srcs], axis=0)
    y = jnp.transpose(x, (1, 0)).astype(jnp.bfloat16)
    lo = jax.lax.bitcast_convert_type(y[:, :NBAND * D // 2],
                                      jnp.uint16).astype(jnp.uint32)
    hi = jax.lax.bitcast_convert_type(y[:, NBAND * D // 2:],
                                      jnp.uint16).astype(jnp.uint32)
    dst_ref[...] = jax.lax.bitcast_convert_type(lo | (hi << 16), jnp.int32)


def _retile(tableT, shift):
    n4 = 1 << shift
    last = (tableT.shape[1] + BLK_W - 1) // BLK_W - 1
    specs = []
    for k in range(NBAND):
        specs.append(pl.BlockSpec(
            (D, BLK_W),
            lambda i, k=k: (0, jnp.minimum((k << shift) // BLK_W + i, last))))
    return pl.pallas_call(
        _retile_body,
        grid=(n4 // BLK_W,),
        in_specs=specs,
        out_specs=pl.BlockSpec((BLK_W, NBAND * D // 2), lambda i: (i, 0)),
        out_shape=jax.ShapeDtypeStruct((n4, NBAND * D // 2), jnp.int32),
    )(*([tableT] * NBAND))


def kernel(user, item, mu, user_bias, item_bias, user_factors, item_factors):
    user2 = user.reshape(NW * NCHUNK, CHUNK)
    item2 = item.reshape(NW * NCHUNK, CHUNK)
    ufac4 = _retile(user_factors.T, USHIFT)
    ifac4 = _retile(item_factors.T, ISHIFT)
    mu16 = jnp.broadcast_to(mu, (L,))
    return _mf(user2, item2, mu16, user_bias, item_bias, ufac4, ifac4)


# speedup vs baseline: 1.0169x; 1.0169x over previous
import jax
import jax.numpy as jnp
from jax import lax
from jax.experimental import pallas as pl
from jax.experimental.pallas import tpu as pltpu
from jax.experimental.pallas import tpu_sc as plsc

BATCH = 16384
D = 32
NC = 2
NS = 16
NW = NC * NS
BPW = BATCH // NW
L = 16
CHUNK = 128
NCHUNK = BPW // CHUNK


def _mf_body(user_hbm, item_hbm, mu_hbm, ub_hbm, ib_hbm, ufac_hbm, ifac_hbm,
             out_hbm, uidx, iidx, umac, imac, uf, itf, ub, ib, outv, muv,
             sem0, sem1):
    wid = lax.axis_index("s") * NC + lax.axis_index("c")
    base_row = wid * NCHUNK
    sems = (sem0, sem1)

    pltpu.sync_copy(user_hbm.at[pl.ds(base_row, NCHUNK)], uidx)
    pltpu.sync_copy(item_hbm.at[pl.ds(base_row, NCHUNK)], iidx)
    pltpu.sync_copy(mu_hbm, muv)

    for j in range(NCHUNK):
        for g in range(CHUNK // L):
            s = pl.ds(g * L, L)
            umac[j, s] = jnp.bitwise_and(uidx[j, s], (1 << USHIFT) - 1)
            imac[j, s] = jnp.bitwise_and(iidx[j, s], (1 << ISHIFT) - 1)

    def start(j):
        slot = j % 2
        cu = pltpu.async_copy(ufac_hbm.at[umac.at[j]],
                              uf.at[pl.ds(slot * CHUNK, CHUNK)], sems[slot])
        ci = pltpu.async_copy(ifac_hbm.at[imac.at[j]],
                              itf.at[pl.ds(slot * CHUNK, CHUNK)], sems[slot])
        cb = pltpu.async_copy(ub_hbm.at[uidx.at[j]],
                              ub.at[pl.ds(j * CHUNK, CHUNK)], sems[slot])
        db = pltpu.async_copy(ib_hbm.at[iidx.at[j]],
                              ib.at[pl.ds(j * CHUNK, CHUNK)], sems[slot])
        return (cu, ci, cb, db)

    mu_v = muv[...]
    inflight = start(0)
    for j in range(NCHUNK):
        for c in inflight:
            c.wait()
        if j + 1 < NCHUNK:
            nxt = start(j + 1)
        slot = j % 2

        def gbody(g, carry):
            rv = slot * CHUNK + g * L + lax.iota(jnp.int32, L)
            s = pl.ds(g * L, L)
            cu = jnp.left_shift(jnp.right_shift(uidx[j, s], USHIFT), 5)
            ci = jnp.left_shift(jnp.right_shift(iidx[j, s], ISHIFT), 5)
            ju = jnp.bitwise_and(cu, 127)
            ji = jnp.bitwise_and(ci, 127)
            shu = jnp.left_shift(jnp.right_shift(cu, 7), 4)
            shi = jnp.left_shift(jnp.right_shift(ci, 7), 4)
            o = pl.ds(j * CHUNK + g * L, L)
            acc = ub[o] + ib[o] + mu_v
            for f in range(D):
                wu = plsc.load_gather(uf, [rv, ju + f])
                wi = plsc.load_gather(itf, [rv, ji + f])
                vu = plsc.bitcast(
                    jnp.left_shift(jnp.right_shift(wu, shu), 16), jnp.float32)
                vi = plsc.bitcast(
                    jnp.left_shift(jnp.right_shift(wi, shi), 16), jnp.float32)
                acc = acc + vu * vi
            outv[o] = acc
            return carry

        lax.fori_loop(0, CHUNK // L, gbody, 0)
        if j + 1 < NCHUNK:
            inflight = nxt

    pltpu.sync_copy(outv, out_hbm.at[pl.ds(wid * BPW, BPW)])


@jax.jit
def _mf(user2, item2, mu, ub_t, ib_t, ufac4, ifac4):
    mesh = plsc.VectorSubcoreMesh(core_axis_name="c", subcore_axis_name="s")
    f = pl.kernel(
        _mf_body,
        out_type=jax.ShapeDtypeStruct((BATCH,), jnp.float32),
        mesh=mesh,
        compiler_params=pltpu.CompilerParams(
            needs_layout_passes=False, use_tc_tiling_on_sc=False),
        scratch_types=[
            pltpu.VMEM((NCHUNK, CHUNK), jnp.int32),
            pltpu.VMEM((NCHUNK, CHUNK), jnp.int32),
            pltpu.VMEM((NCHUNK, CHUNK), jnp.int32),
            pltpu.VMEM((NCHUNK, CHUNK), jnp.int32),
            pltpu.VMEM((2 * CHUNK, 4 * D), jnp.int32),
            pltpu.VMEM((2 * CHUNK, 4 * D), jnp.int32),
            pltpu.VMEM((BPW,), jnp.float32),
            pltpu.VMEM((BPW,), jnp.float32),
            pltpu.VMEM((BPW,), jnp.float32),
            pltpu.VMEM((L,), jnp.float32),
            pltpu.SemaphoreType.DMA,
            pltpu.SemaphoreType.DMA,
        ],
    )
    return f(user2, item2, mu, ub_t, ib_t, ufac4, ifac4)


BLK_W = 8192
NBAND = 8
USHIFT = 17
ISHIFT = 14


def _retile_body(*refs):
    srcs, dst_ref = refs[:-1], refs[-1]
    x = jnp.concatenate([s[...] for s in srcs], axis=0)
    y = jnp.transpose(x, (1, 0)).astype(jnp.bfloat16)
    lo = jax.lax.bitcast_convert_type(y[:, :NBAND * D // 2],
                                      jnp.uint16).astype(jnp.uint32)
    hi = jax.lax.bitcast_convert_type(y[:, NBAND * D // 2:],
                                      jnp.uint16).astype(jnp.uint32)
    dst_ref[...] = jax.lax.bitcast_convert_type(lo | (hi << 16), jnp.int32)


def _retile(tableT, shift):
    n4 = 1 << shift
    last = (tableT.shape[1] + BLK_W - 1) // BLK_W - 1
    specs = []
    for k in range(NBAND):
        specs.append(pl.BlockSpec(
            (D, BLK_W),
            lambda i, k=k: (0, jnp.minimum((k << shift) // BLK_W + i, last))))
    return pl.pallas_call(
        _retile_body,
        grid=(n4 // BLK_W,),
        in_specs=specs,
        out_specs=pl.BlockSpec((BLK_W, NBAND * D // 2), lambda i: (i, 0)),
        out_shape=jax.ShapeDtypeStruct((n4, NBAND * D // 2), jnp.int32),
    )(*([tableT] * NBAND))


def kernel(user, item, mu, user_bias, item_bias, user_factors, item_factors):
    user2 = user.reshape(NW * NCHUNK, CHUNK)
    item2 = item.reshape(NW * NCHUNK, CHUNK)
    ufac4 = _retile(user_factors.T, USHIFT)
    ifac4 = _retile(item_factors.T, ISHIFT)
    mu16 = jnp.broadcast_to(mu, (L,))
    return _mf(user2, item2, mu16, user_bias, item_bias, ufac4, ifac4)
